# baseline (speedup 1.0000x reference)
import functools

import jax
import jax.numpy as jnp
from jax import lax
from jax.experimental import pallas as pl
from jax.experimental.pallas import tpu as pltpu
from jax.experimental.pallas import tpu_sc as plsc

NC = 2
NS = 16
L = 16
NW = NC * NS

B = 16384
D = 64
BPW = B // NW
CHUNK = 128
NCHUNK = BPW // CHUNK
CGROUPS = CHUNK // L


def _rsqrt(x):
    i = plsc.bitcast(x, jnp.int32)
    i = jnp.int32(0x5F3759DF) - lax.shift_right_logical(i, 1)
    y = plsc.bitcast(i, jnp.float32)
    for _ in range(3):
        y = y * (1.5 - 0.5 * x * y * y)
    return y


@functools.partial(
    pl.kernel,
    out_type=jax.ShapeDtypeStruct((B,), jnp.float32),
    mesh=plsc.VectorSubcoreMesh(core_axis_name="c", subcore_axis_name="s"),
    compiler_params=pltpu.CompilerParams(
        needs_layout_passes=False, use_tc_tiling_on_sc=True),
    scratch_types=[
        pltpu.VMEM((BPW,), jnp.int32),
        pltpu.VMEM((BPW,), jnp.int32),
        pltpu.VMEM((BPW,), jnp.int32),
        pltpu.VMEM((BPW,), jnp.int32),
        pltpu.VMEM((BPW,), jnp.int32),
        pltpu.VMEM((BPW,), jnp.int32),
        pltpu.VMEM((CHUNK, 2 * D), jnp.float32),
        pltpu.VMEM((CHUNK, 2 * D), jnp.float32),
        pltpu.VMEM((CHUNK, 2 * D), jnp.float32),
        pltpu.VMEM((BPW,), jnp.float32),
        pltpu.SemaphoreType.DMA,
    ],
)
def _distmult_sc(heads_hbm, relations_hbm, tails_hbm, ent_hbm, rel_hbm,
                 out_hbm, hidx, ridx, tidx, hrow, rrow, trow,
                 e1v, rv, e2v, outv, sem):
    wid = lax.axis_index("s") * NC + lax.axis_index("c")
    base = wid * BPW

    pltpu.sync_copy(heads_hbm.at[pl.ds(base, BPW)], hidx)
    pltpu.sync_copy(relations_hbm.at[pl.ds(base, BPW)], ridx)
    pltpu.sync_copy(tails_hbm.at[pl.ds(base, BPW)], tidx)

    def halve(i, carry):
        s = pl.ds(i * L, L)
        hrow[s] = lax.shift_right_logical(hidx[s], 1)
        rrow[s] = lax.shift_right_logical(ridx[s], 1)
        trow[s] = lax.shift_right_logical(tidx[s], 1)
        return carry

    lax.fori_loop(0, BPW // L, halve, 0)

    def chunk_body(c, carry):
        cbase = c * CHUNK
        c1 = pltpu.async_copy(ent_hbm.at[hrow.at[pl.ds(cbase, CHUNK)]], e1v, sem)
        c2 = pltpu.async_copy(rel_hbm.at[rrow.at[pl.ds(cbase, CHUNK)]], rv, sem)
        c3 = pltpu.async_copy(ent_hbm.at[trow.at[pl.ds(cbase, CHUNK)]], e2v, sem)
        c1.wait()
        c2.wait()
        c3.wait()

        def group(g, gcarry):
            rows = g * L + lax.iota(jnp.int32, L)
            s = pl.ds(cbase + g * L, L)
            cb_h = (hidx[s] & 1) * D
            cb_r = (ridx[s] & 1) * D
            cb_t = (tidx[s] & 1) * D
            acc_d = jnp.zeros((L,), jnp.float32)
            acc_n1 = jnp.zeros((L,), jnp.float32)
            acc_n2 = jnp.zeros((L,), jnp.float32)
            for k in range(D):
                a = plsc.load_gather(e1v, [rows, cb_h + k])
                r_ = plsc.load_gather(rv, [rows, cb_r + k])
                b = plsc.load_gather(e2v, [rows, cb_t + k])
                acc_d = acc_d + a * r_ * b
                acc_n1 = acc_n1 + a * a
                acc_n2 = acc_n2 + b * b
            inv1 = _rsqrt(jnp.maximum(acc_n1, 1e-24))
            inv2 = _rsqrt(jnp.maximum(acc_n2, 1e-24))
            outv[s] = acc_d * inv1 * inv2
            return gcarry

        lax.fori_loop(0, CGROUPS, group, 0)
        return carry

    lax.fori_loop(0, NCHUNK, chunk_body, 0)

    pltpu.sync_copy(outv, out_hbm.at[pl.ds(base, BPW)])


def kernel(heads, relations, tails, entity_embedding, relation_embedding):
    n_ent, d = entity_embedding.shape
    n_rel, _ = relation_embedding.shape
    ent2 = entity_embedding.reshape(n_ent // 2, 2 * d)
    rel2 = relation_embedding.reshape(n_rel // 2, 2 * d)
    return _distmult_sc(
        heads.astype(jnp.int32),
        relations.astype(jnp.int32),
        tails.astype(jnp.int32),
        ent2,
        rel2,
    )


# speedup vs baseline: 1.1051x; 1.1051x over previous
import functools

import jax
import jax.numpy as jnp
from jax import lax
from jax.experimental import pallas as pl
from jax.experimental.pallas import tpu as pltpu
from jax.experimental.pallas import tpu_sc as plsc

NC = 2
NS = 16
L = 16
NW = NC * NS

B = 16384
D = 64
BPW = B // NW
CHUNK = 128
NCHUNK = BPW // CHUNK
CGROUPS = CHUNK // L


def _rsqrt(x):
    i = plsc.bitcast(x, jnp.int32)
    i = jnp.int32(0x5F3759DF) - lax.shift_right_logical(i, 1)
    y = plsc.bitcast(i, jnp.float32)
    for _ in range(3):
        y = y * (1.5 - 0.5 * x * y * y)
    return y


@functools.partial(
    pl.kernel,
    out_type=jax.ShapeDtypeStruct((B,), jnp.float32),
    mesh=plsc.VectorSubcoreMesh(core_axis_name="c", subcore_axis_name="s"),
    compiler_params=pltpu.CompilerParams(
        needs_layout_passes=False, use_tc_tiling_on_sc=True),
    scratch_types=[
        pltpu.VMEM((BPW,), jnp.int32),
        pltpu.VMEM((BPW,), jnp.int32),
        pltpu.VMEM((BPW,), jnp.int32),
        pltpu.VMEM((CHUNK, 2 * D), jnp.float32),
        pltpu.VMEM((CHUNK, 2 * D), jnp.float32),
        pltpu.VMEM((CHUNK, 2 * D), jnp.float32),
        pltpu.VMEM((BPW,), jnp.float32),
        pltpu.SemaphoreType.DMA,
    ],
)
def _distmult_sc(heads_hbm, relations_hbm, tails_hbm, ent_hbm, rel_hbm,
                 out_hbm, hidx, ridx, tidx, e1v, rv, e2v, outv, sem):
    wid = lax.axis_index("s") * NC + lax.axis_index("c")
    base = wid * BPW

    pltpu.sync_copy(heads_hbm.at[pl.ds(base, BPW)], hidx)
    pltpu.sync_copy(relations_hbm.at[pl.ds(base, BPW)], ridx)
    pltpu.sync_copy(tails_hbm.at[pl.ds(base, BPW)], tidx)

    def chunk_body(c, carry):
        cbase = c * CHUNK
        c1 = pltpu.async_copy(ent_hbm.at[hidx.at[pl.ds(cbase, CHUNK)]], e1v, sem)
        c2 = pltpu.async_copy(rel_hbm.at[ridx.at[pl.ds(cbase, CHUNK)]], rv, sem)
        c3 = pltpu.async_copy(ent_hbm.at[tidx.at[pl.ds(cbase, CHUNK)]], e2v, sem)
        c1.wait()
        c2.wait()
        c3.wait()

        def group(g, gcarry):
            rows = g * L + lax.iota(jnp.int32, L)
            acc_d = jnp.zeros((L,), jnp.float32)
            acc_n1 = jnp.zeros((L,), jnp.float32)
            acc_n2 = jnp.zeros((L,), jnp.float32)
            for k in range(D):
                kk = jnp.full((L,), k, jnp.int32)
                a = plsc.load_gather(e1v, [rows, kk])
                r_ = plsc.load_gather(rv, [rows, kk])
                b = plsc.load_gather(e2v, [rows, kk])
                acc_d = acc_d + a * r_ * b
                acc_n1 = acc_n1 + a * a
                acc_n2 = acc_n2 + b * b
            inv1 = _rsqrt(jnp.maximum(acc_n1, 1e-24))
            inv2 = _rsqrt(jnp.maximum(acc_n2, 1e-24))
            outv[pl.ds(cbase + g * L, L)] = acc_d * inv1 * inv2
            return gcarry

        lax.fori_loop(0, CGROUPS, group, 0)
        return carry

    lax.fori_loop(0, NCHUNK, chunk_body, 0)

    pltpu.sync_copy(outv, out_hbm.at[pl.ds(base, BPW)])


def kernel(heads, relations, tails, entity_embedding, relation_embedding):
    n_ent, d = entity_embedding.shape
    n_rel, _ = relation_embedding.shape
    ent2 = jnp.pad(entity_embedding, ((0, 0), (0, 2 * d - entity_embedding.shape[1])))
    rel2 = jnp.pad(relation_embedding, ((0, 0), (0, 2 * d - relation_embedding.shape[1])))
    return _distmult_sc(
        heads.astype(jnp.int32),
        relations.astype(jnp.int32),
        tails.astype(jnp.int32),
        ent2,
        rel2,
    )
